# baseline (speedup 1.0000x reference)
import jax
import jax.numpy as jnp
from jax.experimental import pallas as pl
from jax.experimental.pallas import tpu as pltpu

_INPUT = 2048
_HIDDEN = 16384
_K = 32
_B = 32
_BLK = 512
_NBLK = _HIDDEN // _BLK
_LT = 128
_NT = _HIDDEN // _LT
_TPB = _BLK // _LT
_NSLOT = 8


def _body(x_ref, we_ref, b_ref, wd_ref, o_ref,
          h_s, hs_k, cv_s, cg_s, t_ref, it_ref):
    i = pl.program_id(0)

    @pl.when(i < _NBLK)
    def _encode():
        acc = jax.lax.dot_general(
            x_ref[...], we_ref[...],
            (((1,), (1,)), ((), ())),
            preferred_element_type=jnp.float32,
        )
        blk = jnp.maximum(acc + b_ref[...], 0.0)
        for k in range(_TPB):
            h_s[pl.ds(i * _TPB + k, 1)] = blk[:, k * _LT:(k + 1) * _LT][None]

        iota_c2 = jax.lax.broadcasted_iota(jnp.int32, (_B, _LT), 1)
        iota_s = jax.lax.broadcasted_iota(jnp.int32, (_NSLOT, _B, _LT), 0)

        @pl.when(i == 0)
        def _cand_init():
            cv_s[...] = jnp.full((_NSLOT, _B, _LT), -1.0, jnp.float32)
            cg_s[...] = jnp.full((_NSLOT, _B, _LT), _HIDDEN, jnp.int32)

        cv = cv_s[...]
        cg = cg_s[...]
        for k in range(_TPB):
            v = blk[:, k * _LT:(k + 1) * _LT]
            g = (i * _TPB + k) * _LT + iota_c2
            minv = jnp.min(cv, axis=0)
            ksl = jnp.min(jnp.where(cv == minv[None], iota_s, _NSLOT),
                          axis=0)
            sel = (iota_s == ksl[None]) & (v > minv)[None]
            cv = jnp.where(sel, v[None], cv)
            cg = jnp.where(sel, g[None], cg)
        cv_s[...] = cv
        cg_s[...] = cg

    @pl.when(i == _NBLK)
    def _topk():
        hk = h_s[...]
        iota_k = jax.lax.broadcasted_iota(jnp.int32, (_NT, _B, _LT), 0)
        iota_c = jax.lax.broadcasted_iota(jnp.int32, (_NT, _B, _LT), 2)
        jglob = iota_k * _LT + iota_c

        C = cv_s[...]
        G = cg_s[...]

        def cbody(j, carry):
            Cv, Cg = carry
            m = jnp.max(Cv, axis=(0, 2), keepdims=True)
            gi = jnp.min(jnp.where(Cv == m, Cg, _HIDDEN),
                         axis=(0, 2), keepdims=True)
            Cv = jnp.where((Cv == m) & (Cg == gi), -1.0, Cv)
            t_ref[...] = m.reshape(_B, 1)
            it_ref[...] = gi.reshape(_B, 1)
            return (Cv, Cg)

        jax.lax.fori_loop(0, _K, cbody, (C, G))

        tb = t_ref[...][None]
        ib = it_ref[...][None]
        keep = (hk > tb) | ((hk == tb) & (jglob <= ib))
        cnt = jnp.sum(keep.astype(jnp.int32), axis=(0, 2))
        bad = jnp.any(cnt != _K)

        @pl.when(bad)
        def _fallback():
            hs_k[...] = hk

            def body(j, carry):
                hw = hs_k[...]
                m = jnp.max(hw, axis=(0, 2), keepdims=True)
                im = jnp.min(jnp.where(hw == m, jglob, _HIDDEN),
                             axis=(0, 2), keepdims=True)
                hs_k[...] = jnp.where(jglob == im, -1.0, hw)
                t_ref[...] = m.reshape(_B, 1)
                it_ref[...] = im.reshape(_B, 1)
                return carry

            jax.lax.fori_loop(0, _K, body, 0)

        tb = t_ref[...][None]
        ib = it_ref[...][None]
        keep = (hk > tb) | ((hk == tb) & (jglob <= ib))
        hs_k[...] = jnp.where(keep, hk, 0.0)

    @pl.when(i >= _NBLK)
    def _decode():
        j = i - _NBLK
        parts = [hs_k[pl.ds(j * _TPB + k, 1)].reshape(_B, _LT)
                 for k in range(_TPB)]
        hs = jnp.concatenate(parts, axis=1)
        acc = jax.lax.dot_general(
            hs, wd_ref[...],
            (((1,), (1,)), ((), ())),
            preferred_element_type=jnp.float32,
        )

        @pl.when(i == _NBLK)
        def _init():
            o_ref[...] = acc

        @pl.when(i > _NBLK)
        def _acc():
            o_ref[...] += acc


def kernel(x, W_enc, b_enc, W_dec):
    b2 = b_enc.reshape(1, _HIDDEN)

    recon = pl.pallas_call(
        _body,
        grid=(2 * _NBLK,),
        in_specs=[
            pl.BlockSpec((_B, _INPUT), lambda i: (0, 0)),
            pl.BlockSpec((_BLK, _INPUT),
                         lambda i: (jnp.minimum(i, _NBLK - 1), 0)),
            pl.BlockSpec((1, _BLK),
                         lambda i: (0, jnp.minimum(i, _NBLK - 1))),
            pl.BlockSpec((_INPUT, _BLK),
                         lambda i: (0, jnp.maximum(i - _NBLK, 0))),
        ],
        out_specs=pl.BlockSpec((_B, _INPUT), lambda i: (0, 0)),
        out_shape=jax.ShapeDtypeStruct((_B, _INPUT), jnp.float32),
        scratch_shapes=[
            pltpu.VMEM((_NT, _B, _LT), jnp.float32),
            pltpu.VMEM((_NT, _B, _LT), jnp.float32),
            pltpu.VMEM((_NSLOT, _B, _LT), jnp.float32),
            pltpu.VMEM((_NSLOT, _B, _LT), jnp.int32),
            pltpu.VMEM((_B, 1), jnp.float32),
            pltpu.VMEM((_B, 1), jnp.int32),
        ],
    )(x, W_enc, b2, W_dec)

    return recon


# speedup vs baseline: 1.1669x; 1.1669x over previous
import jax
import jax.numpy as jnp
from jax.experimental import pallas as pl
from jax.experimental.pallas import tpu as pltpu

_INPUT = 2048
_HIDDEN = 16384
_K = 32
_B = 32
_BLK = 1024
_NBLK = _HIDDEN // _BLK
_LT = 128
_NT = _HIDDEN // _LT
_TPB = _BLK // _LT
_NSLOT = 8


def _body(x_ref, we_ref, b_ref, wd_ref, o_ref,
          h_s, hs_k, cv_s, cg_s, t_ref, it_ref):
    i = pl.program_id(0)

    @pl.when(i < _NBLK)
    def _encode():
        acc = jax.lax.dot_general(
            x_ref[...], we_ref[...],
            (((1,), (1,)), ((), ())),
            preferred_element_type=jnp.float32,
        )
        blk = jnp.maximum(acc + b_ref[...], 0.0)
        for k in range(_TPB):
            h_s[pl.ds(i * _TPB + k, 1)] = blk[:, k * _LT:(k + 1) * _LT][None]

        iota_c2 = jax.lax.broadcasted_iota(jnp.int32, (_B, _LT), 1)
        iota_s = jax.lax.broadcasted_iota(jnp.int32, (_NSLOT, _B, _LT), 0)

        @pl.when(i == 0)
        def _cand_init():
            cv_s[...] = jnp.full((_NSLOT, _B, _LT), -1.0, jnp.float32)
            cg_s[...] = jnp.full((_NSLOT, _B, _LT), _HIDDEN, jnp.int32)

        cv = cv_s[...]
        cg = cg_s[...]
        for k in range(_TPB):
            v = blk[:, k * _LT:(k + 1) * _LT]
            g = (i * _TPB + k) * _LT + iota_c2
            minv = jnp.min(cv, axis=0)
            ksl = jnp.min(jnp.where(cv == minv[None], iota_s, _NSLOT),
                          axis=0)
            sel = (iota_s == ksl[None]) & (v > minv)[None]
            cv = jnp.where(sel, v[None], cv)
            cg = jnp.where(sel, g[None], cg)
        cv_s[...] = cv
        cg_s[...] = cg

    @pl.when(i == _NBLK)
    def _topk():
        hk = h_s[...]
        iota_k = jax.lax.broadcasted_iota(jnp.int32, (_NT, _B, _LT), 0)
        iota_c = jax.lax.broadcasted_iota(jnp.int32, (_NT, _B, _LT), 2)
        jglob = iota_k * _LT + iota_c

        C = cv_s[...]
        G = cg_s[...]

        def cbody(j, carry):
            Cv, Cg = carry
            m = jnp.max(Cv, axis=(0, 2), keepdims=True)
            gi = jnp.min(jnp.where(Cv == m, Cg, _HIDDEN),
                         axis=(0, 2), keepdims=True)
            Cv = jnp.where((Cv == m) & (Cg == gi), -1.0, Cv)
            t_ref[...] = m.reshape(_B, 1)
            it_ref[...] = gi.reshape(_B, 1)
            return (Cv, Cg)

        jax.lax.fori_loop(0, _K, cbody, (C, G))

        tb = t_ref[...][None]
        ib = it_ref[...][None]
        keep = (hk > tb) | ((hk == tb) & (jglob <= ib))
        cnt = jnp.sum(keep.astype(jnp.int32), axis=(0, 2))
        bad = jnp.any(cnt != _K)

        @pl.when(jnp.logical_not(bad))
        def _premask():
            hs_k[...] = jnp.where(keep, hk, 0.0)

        @pl.when(bad)
        def _fallback():
            hs_k[...] = hk

            def body(j, carry):
                hw = hs_k[...]
                m = jnp.max(hw, axis=(0, 2), keepdims=True)
                im = jnp.min(jnp.where(hw == m, jglob, _HIDDEN),
                             axis=(0, 2), keepdims=True)
                hs_k[...] = jnp.where(jglob == im, -1.0, hw)
                t_ref[...] = m.reshape(_B, 1)
                it_ref[...] = im.reshape(_B, 1)
                return carry

            jax.lax.fori_loop(0, _K, body, 0)
            tb2 = t_ref[...][None]
            ib2 = it_ref[...][None]
            keep2 = (hk > tb2) | ((hk == tb2) & (jglob <= ib2))
            hs_k[...] = jnp.where(keep2, hk, 0.0)

    @pl.when(i >= _NBLK)
    def _decode():
        j = i - _NBLK
        parts = [hs_k[pl.ds(j * _TPB + k, 1)].reshape(_B, _LT)
                 for k in range(_TPB)]
        hs = jnp.concatenate(parts, axis=1)
        acc = jax.lax.dot_general(
            hs, wd_ref[...],
            (((1,), (1,)), ((), ())),
            preferred_element_type=jnp.float32,
        )

        @pl.when(i == _NBLK)
        def _init():
            o_ref[...] = acc

        @pl.when(i > _NBLK)
        def _acc():
            o_ref[...] += acc


def kernel(x, W_enc, b_enc, W_dec):
    b2 = b_enc.reshape(1, _HIDDEN)

    recon = pl.pallas_call(
        _body,
        grid=(2 * _NBLK,),
        in_specs=[
            pl.BlockSpec((_B, _INPUT), lambda i: (0, 0)),
            pl.BlockSpec((_BLK, _INPUT),
                         lambda i: (jnp.minimum(i, _NBLK - 1), 0)),
            pl.BlockSpec((1, _BLK),
                         lambda i: (0, jnp.minimum(i, _NBLK - 1))),
            pl.BlockSpec((_INPUT, _BLK),
                         lambda i: (0, jnp.maximum(i - _NBLK, 0))),
        ],
        out_specs=pl.BlockSpec((_B, _INPUT), lambda i: (0, 0)),
        out_shape=jax.ShapeDtypeStruct((_B, _INPUT), jnp.float32),
        scratch_shapes=[
            pltpu.VMEM((_NT, _B, _LT), jnp.float32),
            pltpu.VMEM((_NT, _B, _LT), jnp.float32),
            pltpu.VMEM((_NSLOT, _B, _LT), jnp.float32),
            pltpu.VMEM((_NSLOT, _B, _LT), jnp.int32),
            pltpu.VMEM((_B, 1), jnp.float32),
            pltpu.VMEM((_B, 1), jnp.int32),
        ],
    )(x, W_enc, b2, W_dec)

    return recon


# speedup vs baseline: 1.1696x; 1.0024x over previous
import jax
import jax.numpy as jnp
from jax.experimental import pallas as pl
from jax.experimental.pallas import tpu as pltpu

_INPUT = 2048
_HIDDEN = 16384
_K = 32
_B = 32
_BLK = 1024
_NBLK = _HIDDEN // _BLK
_LT = 128
_NT = _HIDDEN // _LT
_TPB = _BLK // _LT
_NSLOT = 8


def _body(x_ref, we_ref, b_ref, wd_ref, o_ref,
          h_s, hs_k, cv_s, cg_s, t_ref, it_ref):
    i = pl.program_id(0)

    @pl.when(i < _NBLK)
    def _encode():
        acc = jax.lax.dot_general(
            x_ref[...], we_ref[...],
            (((1,), (1,)), ((), ())),
            preferred_element_type=jnp.float32,
        )
        blk = jnp.maximum(acc + b_ref[...], 0.0)
        for k in range(_TPB):
            h_s[pl.ds(i * _TPB + k, 1)] = blk[:, k * _LT:(k + 1) * _LT][None]

        iota_c2 = jax.lax.broadcasted_iota(jnp.int32, (_B, _LT), 1)
        iota_s = jax.lax.broadcasted_iota(jnp.int32, (_NSLOT, _B, _LT), 0)

        @pl.when(i == 0)
        def _cand_init():
            cv_s[...] = jnp.full((_NSLOT, _B, _LT), -1.0, jnp.float32)
            cg_s[...] = jnp.full((_NSLOT, _B, _LT), _HIDDEN, jnp.int32)

        cv = cv_s[...]
        cg = cg_s[...]
        for k in range(_TPB):
            v = blk[:, k * _LT:(k + 1) * _LT]
            g = (i * _TPB + k) * _LT + iota_c2
            minv = jnp.min(cv, axis=0)
            ksl = jnp.min(jnp.where(cv == minv[None], iota_s, _NSLOT),
                          axis=0)
            sel = (iota_s == ksl[None]) & (v > minv)[None]
            cv = jnp.where(sel, v[None], cv)
            cg = jnp.where(sel, g[None], cg)
        cv_s[...] = cv
        cg_s[...] = cg

    @pl.when(i == _NBLK)
    def _topk():
        hk = h_s[...]
        iota_k = jax.lax.broadcasted_iota(jnp.int32, (_NT, _B, _LT), 0)
        iota_c = jax.lax.broadcasted_iota(jnp.int32, (_NT, _B, _LT), 2)
        jglob = iota_k * _LT + iota_c

        C = cv_s[...]
        G = cg_s[...]

        def cbody(j, carry):
            Cv, Cg = carry
            m = jnp.max(Cv, axis=(0, 2), keepdims=True)
            gi = jnp.min(jnp.where(Cv == m, Cg, _HIDDEN),
                         axis=(0, 2), keepdims=True)
            Cv = jnp.where((Cv == m) & (Cg == gi), -1.0, Cv)
            t_ref[...] = m.reshape(_B, 1)
            it_ref[...] = gi.reshape(_B, 1)
            return (Cv, Cg)

        jax.lax.fori_loop(0, _K, cbody, (C, G))

        tb = t_ref[...][None]
        ib = it_ref[...][None]
        keep = (hk > tb) | ((hk == tb) & (jglob <= ib))
        cnt = jnp.sum(keep.astype(jnp.int32), axis=(0, 2))
        bad = jnp.any(cnt != _K)

        @pl.when(bad)
        def _fallback():
            hs_k[...] = hk

            def body(j, carry):
                hw = hs_k[...]
                m = jnp.max(hw, axis=(0, 2), keepdims=True)
                im = jnp.min(jnp.where(hw == m, jglob, _HIDDEN),
                             axis=(0, 2), keepdims=True)
                hs_k[...] = jnp.where(jglob == im, -1.0, hw)
                t_ref[...] = m.reshape(_B, 1)
                it_ref[...] = im.reshape(_B, 1)
                return carry

            jax.lax.fori_loop(0, _K, body, 0)

    @pl.when(i >= _NBLK)
    def _decode():
        j = i - _NBLK
        parts = [h_s[pl.ds(j * _TPB + k, 1)].reshape(_B, _LT)
                 for k in range(_TPB)]
        hraw = jnp.concatenate(parts, axis=1)
        iota = jax.lax.broadcasted_iota(jnp.int32, (_B, _BLK), 1) + j * _BLK
        keepb = (hraw > t_ref[...]) | ((hraw == t_ref[...]) &
                                       (iota <= it_ref[...]))
        hs = jnp.where(keepb, hraw, 0.0)
        acc = jax.lax.dot_general(
            hs, wd_ref[...],
            (((1,), (1,)), ((), ())),
            preferred_element_type=jnp.float32,
        )

        @pl.when(i == _NBLK)
        def _init():
            o_ref[...] = acc

        @pl.when(i > _NBLK)
        def _acc():
            o_ref[...] += acc


def kernel(x, W_enc, b_enc, W_dec):
    b2 = b_enc.reshape(1, _HIDDEN)

    recon = pl.pallas_call(
        _body,
        grid=(2 * _NBLK,),
        in_specs=[
            pl.BlockSpec((_B, _INPUT), lambda i: (0, 0)),
            pl.BlockSpec((_BLK, _INPUT),
                         lambda i: (jnp.minimum(i, _NBLK - 1), 0)),
            pl.BlockSpec((1, _BLK),
                         lambda i: (0, jnp.minimum(i, _NBLK - 1))),
            pl.BlockSpec((_INPUT, _BLK),
                         lambda i: (0, jnp.maximum(i - _NBLK, 0))),
        ],
        out_specs=pl.BlockSpec((_B, _INPUT), lambda i: (0, 0)),
        out_shape=jax.ShapeDtypeStruct((_B, _INPUT), jnp.float32),
        scratch_shapes=[
            pltpu.VMEM((_NT, _B, _LT), jnp.float32),
            pltpu.VMEM((_NT, _B, _LT), jnp.float32),
            pltpu.VMEM((_NSLOT, _B, _LT), jnp.float32),
            pltpu.VMEM((_NSLOT, _B, _LT), jnp.int32),
            pltpu.VMEM((_B, 1), jnp.float32),
            pltpu.VMEM((_B, 1), jnp.int32),
        ],
    )(x, W_enc, b2, W_dec)

    return recon
